# baseline (speedup 1.0000x reference)
import jax
import jax.numpy as jnp
from jax.experimental import pallas as pl
from jax.experimental.pallas import tpu as pltpu

_NEG = -1.0e9
_BOT = -3.0e38
_TOP_P = 0.9
_K = 50

_VOCAB = 100000
_LANE = 128
_VPAD = 102400
_NCHUNK = _VPAD // _LANE
_TILE_C = 32
_TILE = _TILE_C * _LANE
_NT = _NCHUNK // _TILE_C
_NSTEPS = 2 * _NT + 1
_BIG_I = 2**30


def _topk_vals(a3):
    cm = jnp.max(a3, axis=2)
    ii_c = jax.lax.broadcasted_iota(jnp.int32, (8, _NCHUNK), 1)
    ii_k = jax.lax.broadcasted_iota(jnp.int32, (8, _K), 1)

    def chunk_step(j, carry):
        cm, cidx = carry
        m = jnp.max(cm, axis=1, keepdims=True)
        pos = jnp.min(jnp.where(cm == m, ii_c, _BIG_I), axis=1, keepdims=True)
        cm = jnp.where(ii_c == pos, _BOT, cm)
        cidx = cidx + pos * (ii_k == j)
        return cm, cidx

    _, cidx = jax.lax.fori_loop(
        0, _K, chunk_step, (cm, jnp.zeros((8, _K), jnp.int32)))

    smat = (cidx[:, :, None] ==
            jax.lax.broadcasted_iota(jnp.int32, (8, _K, _NCHUNK), 2)
            ).astype(jnp.float32)
    cand = jax.lax.dot_general(
        smat, a3, (((2,), (1,)), ((0,), (0,))),
        precision=jax.lax.Precision.HIGHEST,
        preferred_element_type=jnp.float32)

    ii_3 = (jax.lax.broadcasted_iota(jnp.int32, (8, _K, _LANE), 1) * _LANE +
            jax.lax.broadcasted_iota(jnp.int32, (8, _K, _LANE), 2))

    def val_step(j, carry):
        cand, vals = carry
        m = jnp.max(cand, axis=(1, 2))
        pos = jnp.min(jnp.where(cand == m[:, None, None], ii_3, _BIG_I),
                      axis=(1, 2))
        cand = jnp.where(ii_3 == pos[:, None, None], _BOT, cand)
        vals = vals + m[:, None] * (ii_k == j)
        return cand, vals

    _, vals = jax.lax.fori_loop(
        0, _K, val_step, (cand, jnp.zeros((8, _K), jnp.float32)))
    return vals


def _phase_a(i, h_ref, e_ref, t_ref, b_ref, l_sc, vb_sc):
    lg = jax.lax.dot_general(
        h_ref[...], e_ref[...], (((1,), (1,)), ((), ())),
        preferred_element_type=jnp.float32)
    lg = lg / t_ref[...]
    col = i * _TILE + jax.lax.broadcasted_iota(jnp.int32, (8, _TILE), 1)
    lg = jnp.where(col < _VOCAB, lg, _BOT)
    l_sc[:, pl.ds(i * _TILE_C, _TILE_C), :] = lg.reshape(8, _TILE_C, _LANE)

    @pl.when(i == 0)
    def _():
        vb_sc[...] = _topk_vals(b_ref[...])


def _phase_b(koh_ref, l_sc, vb_sc, p_sc):
    vals = jnp.concatenate([_topk_vals(l_sc[...]), vb_sc[...]],
                           axis=0)
    ii_k = jax.lax.broadcasted_iota(jnp.int32, (16, _K), 1)

    kth = jnp.sum(vals * koh_ref[...], axis=1, keepdims=True)
    vm = jnp.where(vals < kth, _NEG, vals)

    m0 = vals[:, 0:1]
    e = jnp.exp(vm - m0)
    s = jnp.sum(e, axis=1, keepdims=True)
    p = e / s
    tri = (jax.lax.broadcasted_iota(jnp.int32, (_K, _K), 0) <=
           jax.lax.broadcasted_iota(jnp.int32, (_K, _K), 1)).astype(jnp.float32)
    cum = jax.lax.dot_general(p, tri, (((1,), (0,)), ((), ())),
                              precision=jax.lax.Precision.HIGHEST,
                              preferred_element_type=jnp.float32)
    nkeep = jnp.sum(jnp.where((cum - p) > _TOP_P, 0, 1), axis=1)
    lastk = jnp.maximum(nkeep - 1, 0)
    thr = jnp.sum(vals * (ii_k == lastk[:, None]), axis=1)

    s2 = jnp.sum(jnp.where(vm >= thr[:, None], e, 0.0), axis=1)
    logs = jnp.log(s2)

    ii_l = jax.lax.broadcasted_iota(jnp.int32, (8, _LANE), 1)
    p_sc[...] = (thr[0:8, None] * (ii_l == 0) +
                 m0[0:8] * (ii_l == 1) +
                 logs[0:8, None] * (ii_l == 2) +
                 thr[8:16, None] * (ii_l == 3))


def _phase_c(j, b_ref, probs_ref, lp_ref, ntok_ref, l_sc, p_sc, mx_sc, ix_sc):
    prm = p_sc[...]
    thr_l = prm[:, 0:1, None]
    m0 = prm[:, 1:2, None]
    logs = prm[:, 2:3, None]
    thr_b = prm[:, 3:4, None]

    x = l_sc[:, pl.ds(j * _TILE_C, _TILE_C), :]
    b = b_ref[:, pl.ds(j * _TILE_C, _TILE_C), :]
    xm = jnp.where(x < thr_l, _NEG, x)
    ls = (xm - m0) - logs
    lbm = jnp.where(b < thr_b, _NEG, b)
    lp = ls + lbm
    probs_ref[...] = jnp.exp(lp)
    lp_ref[...] = lp

    col = (j * _TILE +
           jax.lax.broadcasted_iota(jnp.int32, (8, _TILE_C, _LANE), 1) * _LANE +
           jax.lax.broadcasted_iota(jnp.int32, (8, _TILE_C, _LANE), 2))
    tmx = jnp.max(lp, axis=(1, 2))
    tix = jnp.min(jnp.where(lp == tmx[:, None, None], col, _BIG_I),
                  axis=(1, 2))

    @pl.when(j == 0)
    def _():
        mx_sc[...] = jnp.full((8, _LANE), _BOT, jnp.float32)
        ix_sc[...] = jnp.zeros((8, _LANE), jnp.int32)

    better = (tmx[:, None] > mx_sc[...])
    mx_sc[...] = jnp.where(better, tmx[:, None], mx_sc[...])
    ix_sc[...] = jnp.where(better, tix[:, None], ix_sc[...])

    @pl.when(j == _NT - 1)
    def _():
        ntok_ref[...] = ix_sc[...]


def _fused_kernel(h_ref, e_ref, t_ref, b_ref, koh_ref,
                  probs_ref, lp_ref, ntok_ref,
                  l_sc, vb_sc, p_sc, mx_sc, ix_sc):
    i = pl.program_id(0)

    @pl.when(i < _NT)
    def _():
        _phase_a(i, h_ref, e_ref, t_ref, b_ref, l_sc, vb_sc)

    @pl.when(i == _NT)
    def _():
        _phase_b(koh_ref, l_sc, vb_sc, p_sc)

    @pl.when(i > _NT)
    def _():
        _phase_c(i - _NT - 1, b_ref, probs_ref, lp_ref, ntok_ref,
                 l_sc, p_sc, mx_sc, ix_sc)


def kernel(embedding, logprob_bias, hidden_states, temperatures, k):
    f32 = jnp.float32
    bias3 = jnp.pad(logprob_bias, ((0, 0), (0, _VPAD - _VOCAB)),
                    constant_values=_BOT).reshape(8, _NCHUNK, _LANE)
    koh = jnp.broadcast_to(
        (jnp.arange(_K, dtype=jnp.int32) ==
         (jnp.asarray(k, jnp.int32) - 1)).astype(f32)[None, :], (16, _K))

    def e_idx(i):
        return (jnp.minimum(i, _NT - 1), 0)

    def out_idx(i):
        return (0, jnp.clip(i - _NT - 1, 0, _NT - 1), 0)

    probs3, lp3, ntok = pl.pallas_call(
        _fused_kernel,
        grid=(_NSTEPS,),
        in_specs=[
            pl.BlockSpec((8, 1024), lambda i: (0, 0)),
            pl.BlockSpec((_TILE, 1024), e_idx),
            pl.BlockSpec((8, 1), lambda i: (0, 0)),
            pl.BlockSpec((8, _NCHUNK, _LANE), lambda i: (0, 0, 0)),
            pl.BlockSpec((16, _K), lambda i: (0, 0)),
        ],
        out_specs=[
            pl.BlockSpec((8, _TILE_C, _LANE), out_idx),
            pl.BlockSpec((8, _TILE_C, _LANE), out_idx),
            pl.BlockSpec((8, _LANE), lambda i: (0, 0)),
        ],
        out_shape=[
            jax.ShapeDtypeStruct((8, _NCHUNK, _LANE), f32),
            jax.ShapeDtypeStruct((8, _NCHUNK, _LANE), f32),
            jax.ShapeDtypeStruct((8, _LANE), jnp.int32),
        ],
        scratch_shapes=[
            pltpu.VMEM((8, _NCHUNK, _LANE), f32),
            pltpu.VMEM((8, _K), f32),
            pltpu.VMEM((8, _LANE), f32),
            pltpu.VMEM((8, _LANE), f32),
            pltpu.VMEM((8, _LANE), jnp.int32),
        ],
    )(hidden_states, embedding, temperatures.reshape(8, 1), bias3, koh)

    probs = probs3.reshape(8, _VPAD)[:, :_VOCAB]
    logprobs = lp3.reshape(8, _VPAD)[:, :_VOCAB]
    next_tokens = ntok[:, 0]
    return next_tokens, probs, logprobs


# speedup vs baseline: 1.1144x; 1.1144x over previous
import jax
import jax.numpy as jnp
from jax.experimental import pallas as pl
from jax.experimental.pallas import tpu as pltpu

_NEG = -1.0e9
_BOT = -3.0e38
_TOP_P = 0.9
_K = 50

_VOCAB = 100000
_LANE = 128
_VPAD = 102400
_NCHUNK = _VPAD // _LANE
_TILE_C = 32
_TILE = _TILE_C * _LANE
_NT = _NCHUNK // _TILE_C
_NSTEPS = 2 * _NT + 1
_BIG_I = 2**30


def _phase_a(i, h_ref, e_ref, t_ref, l_sc):
    lg = jax.lax.dot_general(
        h_ref[...], e_ref[...], (((1,), (1,)), ((), ())),
        preferred_element_type=jnp.float32)
    lg = lg / t_ref[...]
    col = i * _TILE + jax.lax.broadcasted_iota(jnp.int32, (8, _TILE), 1)
    lg = jnp.where(col < _VOCAB, lg, _BOT)
    l_sc[:, pl.ds(i * _TILE_C, _TILE_C), :] = lg.reshape(8, _TILE_C, _LANE)


def _phase_b(b_ref, koh_ref, l_sc, p_sc):
    a3 = jnp.concatenate([l_sc[...], b_ref[...]], axis=0)
    cm = jnp.max(a3, axis=2)

    ii_c = jax.lax.broadcasted_iota(jnp.int32, (16, _NCHUNK), 1)
    ii_k = jax.lax.broadcasted_iota(jnp.int32, (16, _K), 1)

    def chunk_step(j, carry):
        cm, cidx = carry
        m = jnp.max(cm, axis=1, keepdims=True)
        pos = jnp.min(jnp.where(cm == m, ii_c, _BIG_I), axis=1, keepdims=True)
        cm = jnp.where(ii_c == pos, _BOT, cm)
        cidx = cidx + pos * (ii_k == j)
        return cm, cidx

    _, cidx = jax.lax.fori_loop(
        0, _K, chunk_step,
        (cm, jnp.zeros((16, _K), jnp.int32)))

    smat = (cidx[:, :, None] ==
            jax.lax.broadcasted_iota(jnp.int32, (16, _K, _NCHUNK), 2)
            ).astype(jnp.float32)
    cand = jax.lax.dot_general(
        smat, a3, (((2,), (1,)), ((0,), (0,))),
        precision=jax.lax.Precision.HIGHEST,
        preferred_element_type=jnp.float32)

    ii_3 = (jax.lax.broadcasted_iota(jnp.int32, (16, _K, _LANE), 1) * _LANE +
            jax.lax.broadcasted_iota(jnp.int32, (16, _K, _LANE), 2))

    def val_step(j, carry):
        cand, vals = carry
        m = jnp.max(cand, axis=(1, 2))
        pos = jnp.min(jnp.where(cand == m[:, None, None], ii_3, _BIG_I),
                      axis=(1, 2))
        cand = jnp.where(ii_3 == pos[:, None, None], _BOT, cand)
        vals = vals + m[:, None] * (ii_k == j)
        return cand, vals

    _, vals = jax.lax.fori_loop(
        0, _K, val_step,
        (cand, jnp.zeros((16, _K), jnp.float32)))

    kth = jnp.sum(vals * koh_ref[...], axis=1, keepdims=True)
    vm = jnp.where(vals < kth, _NEG, vals)

    m0 = vals[:, 0:1]
    e = jnp.exp(vm - m0)
    s = jnp.sum(e, axis=1, keepdims=True)
    p = e / s
    tri = (jax.lax.broadcasted_iota(jnp.int32, (_K, _K), 0) <=
           jax.lax.broadcasted_iota(jnp.int32, (_K, _K), 1)).astype(jnp.float32)
    cum = jax.lax.dot_general(p, tri, (((1,), (0,)), ((), ())),
                              precision=jax.lax.Precision.HIGHEST,
                              preferred_element_type=jnp.float32)
    nkeep = jnp.sum(jnp.where((cum - p) > _TOP_P, 0, 1), axis=1)
    lastk = jnp.maximum(nkeep - 1, 0)
    thr = jnp.sum(vals * (ii_k == lastk[:, None]), axis=1)

    s2 = jnp.sum(jnp.where(vm >= thr[:, None], e, 0.0), axis=1)
    logs = jnp.log(s2)

    ii_l = jax.lax.broadcasted_iota(jnp.int32, (8, _LANE), 1)
    p_sc[...] = (thr[0:8, None] * (ii_l == 0) +
                 m0[0:8] * (ii_l == 1) +
                 logs[0:8, None] * (ii_l == 2) +
                 thr[8:16, None] * (ii_l == 3))


def _phase_c(j, b_ref, probs_ref, lp_ref, ntok_ref, l_sc, p_sc, mx_sc, ix_sc):
    prm = p_sc[...]
    thr_l = prm[:, 0:1, None]
    m0 = prm[:, 1:2, None]
    logs = prm[:, 2:3, None]
    thr_b = prm[:, 3:4, None]

    x = l_sc[:, pl.ds(j * _TILE_C, _TILE_C), :]
    b = b_ref[:, pl.ds(j * _TILE_C, _TILE_C), :]
    xm = jnp.where(x < thr_l, _NEG, x)
    ls = (xm - m0) - logs
    lbm = jnp.where(b < thr_b, _NEG, b)
    lp = ls + lbm
    probs_ref[...] = jnp.exp(lp)
    lp_ref[...] = lp

    col = (j * _TILE +
           jax.lax.broadcasted_iota(jnp.int32, (8, _TILE_C, _LANE), 1) * _LANE +
           jax.lax.broadcasted_iota(jnp.int32, (8, _TILE_C, _LANE), 2))
    tmx = jnp.max(lp, axis=(1, 2))
    tix = jnp.min(jnp.where(lp == tmx[:, None, None], col, _BIG_I),
                  axis=(1, 2))

    @pl.when(j == 0)
    def _():
        mx_sc[...] = jnp.full((8, _LANE), _BOT, jnp.float32)
        ix_sc[...] = jnp.zeros((8, _LANE), jnp.int32)

    better = (tmx[:, None] > mx_sc[...])
    mx_sc[...] = jnp.where(better, tmx[:, None], mx_sc[...])
    ix_sc[...] = jnp.where(better, tix[:, None], ix_sc[...])

    @pl.when(j == _NT - 1)
    def _():
        ntok_ref[...] = ix_sc[...]


def _fused_kernel(h_ref, e_ref, t_ref, b_ref, koh_ref,
                  probs_ref, lp_ref, ntok_ref,
                  l_sc, p_sc, mx_sc, ix_sc):
    i = pl.program_id(0)

    @pl.when(i < _NT)
    def _():
        _phase_a(i, h_ref, e_ref, t_ref, l_sc)

    @pl.when(i == _NT)
    def _():
        _phase_b(b_ref, koh_ref, l_sc, p_sc)

    @pl.when(i > _NT)
    def _():
        _phase_c(i - _NT - 1, b_ref, probs_ref, lp_ref, ntok_ref,
                 l_sc, p_sc, mx_sc, ix_sc)


def kernel(embedding, logprob_bias, hidden_states, temperatures, k):
    f32 = jnp.float32
    bias3 = jnp.pad(logprob_bias, ((0, 0), (0, _VPAD - _VOCAB)),
                    constant_values=_BOT).reshape(8, _NCHUNK, _LANE)
    koh = jnp.broadcast_to(
        (jnp.arange(_K, dtype=jnp.int32) ==
         (jnp.asarray(k, jnp.int32) - 1)).astype(f32)[None, :], (16, _K))

    def e_idx(i):
        return (jnp.minimum(i, _NT - 1), 0)

    def out_idx(i):
        return (0, jnp.clip(i - _NT - 1, 0, _NT - 1), 0)

    probs3, lp3, ntok = pl.pallas_call(
        _fused_kernel,
        grid=(_NSTEPS,),
        in_specs=[
            pl.BlockSpec((8, 1024), lambda i: (0, 0)),
            pl.BlockSpec((_TILE, 1024), e_idx),
            pl.BlockSpec((8, 1), lambda i: (0, 0)),
            pl.BlockSpec((8, _NCHUNK, _LANE), lambda i: (0, 0, 0)),
            pl.BlockSpec((16, _K), lambda i: (0, 0)),
        ],
        out_specs=[
            pl.BlockSpec((8, _TILE_C, _LANE), out_idx),
            pl.BlockSpec((8, _TILE_C, _LANE), out_idx),
            pl.BlockSpec((8, _LANE), lambda i: (0, 0)),
        ],
        out_shape=[
            jax.ShapeDtypeStruct((8, _NCHUNK, _LANE), f32),
            jax.ShapeDtypeStruct((8, _NCHUNK, _LANE), f32),
            jax.ShapeDtypeStruct((8, _LANE), jnp.int32),
        ],
        scratch_shapes=[
            pltpu.VMEM((8, _NCHUNK, _LANE), f32),
            pltpu.VMEM((8, _LANE), f32),
            pltpu.VMEM((8, _LANE), f32),
            pltpu.VMEM((8, _LANE), jnp.int32),
        ],
    )(hidden_states, embedding, temperatures.reshape(8, 1), bias3, koh)

    probs = probs3.reshape(8, _VPAD)[:, :_VOCAB]
    logprobs = lp3.reshape(8, _VPAD)[:, :_VOCAB]
    next_tokens = ntok[:, 0]
    return next_tokens, probs, logprobs


# speedup vs baseline: 1.1235x; 1.0082x over previous
import jax
import jax.numpy as jnp
from jax.experimental import pallas as pl
from jax.experimental.pallas import tpu as pltpu

_NEG = -1.0e9
_BOT = -3.0e38
_TOP_P = 0.9
_K = 50

_VOCAB = 100000
_LANE = 128
_VPAD = 102400
_NCHUNK = _VPAD // _LANE
_TILE_C = 40
_TILE = _TILE_C * _LANE
_NT = _NCHUNK // _TILE_C
_NSTEPS = 2 * _NT + 1
_BIG_I = 2**30


def _phase_a(i, h_ref, e_ref, t_ref, l_sc):
    lg = jax.lax.dot_general(
        h_ref[...], e_ref[...], (((1,), (1,)), ((), ())),
        preferred_element_type=jnp.float32)
    lg = lg / t_ref[...]
    col = i * _TILE + jax.lax.broadcasted_iota(jnp.int32, (8, _TILE), 1)
    lg = jnp.where(col < _VOCAB, lg, _BOT)
    l_sc[:, pl.ds(i * _TILE_C, _TILE_C), :] = lg.reshape(8, _TILE_C, _LANE)


def _phase_b(b_ref, koh_ref, l_sc, p_sc):
    a3 = jnp.concatenate([l_sc[...], b_ref[...]], axis=0)
    cm = jnp.max(a3, axis=2)

    ii_c = jax.lax.broadcasted_iota(jnp.int32, (16, _NCHUNK), 1)
    ii_k = jax.lax.broadcasted_iota(jnp.int32, (16, _K), 1)

    def chunk_step(j, carry):
        cm, cidx = carry
        m = jnp.max(cm, axis=1, keepdims=True)
        pos = jnp.min(jnp.where(cm == m, ii_c, _BIG_I), axis=1, keepdims=True)
        cm = jnp.where(ii_c == pos, _BOT, cm)
        cidx = cidx + pos * (ii_k == j)
        return cm, cidx

    _, cidx = jax.lax.fori_loop(
        0, _K, chunk_step,
        (cm, jnp.zeros((16, _K), jnp.int32)))

    smat = (cidx[:, :, None] ==
            jax.lax.broadcasted_iota(jnp.int32, (16, _K, _NCHUNK), 2)
            ).astype(jnp.float32)
    cand = jax.lax.dot_general(
        smat, a3, (((2,), (1,)), ((0,), (0,))),
        precision=jax.lax.Precision.HIGHEST,
        preferred_element_type=jnp.float32)

    ii_3 = (jax.lax.broadcasted_iota(jnp.int32, (16, _K, _LANE), 1) * _LANE +
            jax.lax.broadcasted_iota(jnp.int32, (16, _K, _LANE), 2))

    def val_step(j, carry):
        cand, vals = carry
        m = jnp.max(cand, axis=(1, 2))
        pos = jnp.min(jnp.where(cand == m[:, None, None], ii_3, _BIG_I),
                      axis=(1, 2))
        cand = jnp.where(ii_3 == pos[:, None, None], _BOT, cand)
        vals = vals + m[:, None] * (ii_k == j)
        return cand, vals

    _, vals = jax.lax.fori_loop(
        0, _K, val_step,
        (cand, jnp.zeros((16, _K), jnp.float32)))

    kth = jnp.sum(vals * koh_ref[...], axis=1, keepdims=True)
    vm = jnp.where(vals < kth, _NEG, vals)

    m0 = vals[:, 0:1]
    e = jnp.exp(vm - m0)
    s = jnp.sum(e, axis=1, keepdims=True)
    p = e / s
    tri = (jax.lax.broadcasted_iota(jnp.int32, (_K, _K), 0) <=
           jax.lax.broadcasted_iota(jnp.int32, (_K, _K), 1)).astype(jnp.float32)
    cum = jax.lax.dot_general(p, tri, (((1,), (0,)), ((), ())),
                              precision=jax.lax.Precision.HIGHEST,
                              preferred_element_type=jnp.float32)
    nkeep = jnp.sum(jnp.where((cum - p) > _TOP_P, 0, 1), axis=1)
    lastk = jnp.maximum(nkeep - 1, 0)
    thr = jnp.sum(vals * (ii_k == lastk[:, None]), axis=1)

    s2 = jnp.sum(jnp.where(vm >= thr[:, None], e, 0.0), axis=1)
    logs = jnp.log(s2)

    ii_l = jax.lax.broadcasted_iota(jnp.int32, (8, _LANE), 1)
    p_sc[...] = (thr[0:8, None] * (ii_l == 0) +
                 m0[0:8] * (ii_l == 1) +
                 logs[0:8, None] * (ii_l == 2) +
                 thr[8:16, None] * (ii_l == 3))


def _phase_c(j, b_ref, probs_ref, lp_ref, ntok_ref, l_sc, p_sc, mx_sc, ix_sc):
    prm = p_sc[...]
    thr_l = prm[:, 0:1, None]
    m0 = prm[:, 1:2, None]
    logs = prm[:, 2:3, None]
    thr_b = prm[:, 3:4, None]

    x = l_sc[:, pl.ds(j * _TILE_C, _TILE_C), :]
    b = b_ref[:, pl.ds(j * _TILE_C, _TILE_C), :]
    xm = jnp.where(x < thr_l, _NEG, x)
    ls = (xm - m0) - logs
    lbm = jnp.where(b < thr_b, _NEG, b)
    lp = ls + lbm
    probs_ref[...] = jnp.exp(lp)
    lp_ref[...] = lp

    col = (j * _TILE +
           jax.lax.broadcasted_iota(jnp.int32, (8, _TILE_C, _LANE), 1) * _LANE +
           jax.lax.broadcasted_iota(jnp.int32, (8, _TILE_C, _LANE), 2))
    tmx = jnp.max(lp, axis=(1, 2))
    tix = jnp.min(jnp.where(lp == tmx[:, None, None], col, _BIG_I),
                  axis=(1, 2))

    @pl.when(j == 0)
    def _():
        mx_sc[...] = jnp.full((8, _LANE), _BOT, jnp.float32)
        ix_sc[...] = jnp.zeros((8, _LANE), jnp.int32)

    better = (tmx[:, None] > mx_sc[...])
    mx_sc[...] = jnp.where(better, tmx[:, None], mx_sc[...])
    ix_sc[...] = jnp.where(better, tix[:, None], ix_sc[...])

    @pl.when(j == _NT - 1)
    def _():
        ntok_ref[...] = ix_sc[...]


def _fused_kernel(h_ref, e_ref, t_ref, b_ref, koh_ref,
                  probs_ref, lp_ref, ntok_ref,
                  l_sc, p_sc, mx_sc, ix_sc):
    i = pl.program_id(0)

    @pl.when(i < _NT)
    def _():
        _phase_a(i, h_ref, e_ref, t_ref, l_sc)

    @pl.when(i == _NT)
    def _():
        _phase_b(b_ref, koh_ref, l_sc, p_sc)

    @pl.when(i > _NT)
    def _():
        _phase_c(i - _NT - 1, b_ref, probs_ref, lp_ref, ntok_ref,
                 l_sc, p_sc, mx_sc, ix_sc)


def kernel(embedding, logprob_bias, hidden_states, temperatures, k):
    f32 = jnp.float32
    bias3 = jnp.pad(logprob_bias, ((0, 0), (0, _VPAD - _VOCAB)),
                    constant_values=_BOT).reshape(8, _NCHUNK, _LANE)
    koh = jnp.broadcast_to(
        (jnp.arange(_K, dtype=jnp.int32) ==
         (jnp.asarray(k, jnp.int32) - 1)).astype(f32)[None, :], (16, _K))

    def e_idx(i):
        return (jnp.minimum(i, _NT - 1), 0)

    def out_idx(i):
        return (0, jnp.clip(i - _NT - 1, 0, _NT - 1), 0)

    probs3, lp3, ntok = pl.pallas_call(
        _fused_kernel,
        grid=(_NSTEPS,),
        in_specs=[
            pl.BlockSpec((8, 1024), lambda i: (0, 0)),
            pl.BlockSpec((_TILE, 1024), e_idx),
            pl.BlockSpec((8, 1), lambda i: (0, 0)),
            pl.BlockSpec((8, _NCHUNK, _LANE), lambda i: (0, 0, 0)),
            pl.BlockSpec((16, _K), lambda i: (0, 0)),
        ],
        out_specs=[
            pl.BlockSpec((8, _TILE_C, _LANE), out_idx),
            pl.BlockSpec((8, _TILE_C, _LANE), out_idx),
            pl.BlockSpec((8, _LANE), lambda i: (0, 0)),
        ],
        out_shape=[
            jax.ShapeDtypeStruct((8, _NCHUNK, _LANE), f32),
            jax.ShapeDtypeStruct((8, _NCHUNK, _LANE), f32),
            jax.ShapeDtypeStruct((8, _LANE), jnp.int32),
        ],
        scratch_shapes=[
            pltpu.VMEM((8, _NCHUNK, _LANE), f32),
            pltpu.VMEM((8, _LANE), f32),
            pltpu.VMEM((8, _LANE), f32),
            pltpu.VMEM((8, _LANE), jnp.int32),
        ],
    )(hidden_states, embedding, temperatures.reshape(8, 1), bias3, koh)

    probs = probs3.reshape(8, _VPAD)[:, :_VOCAB]
    logprobs = lp3.reshape(8, _VPAD)[:, :_VOCAB]
    next_tokens = ntok[:, 0]
    return next_tokens, probs, logprobs
